# baseline (speedup 1.0000x reference)
import functools

import jax
import jax.numpy as jnp
from jax import lax
from jax.experimental import pallas as pl
from jax.experimental.pallas import tpu as pltpu

_DROPOUT_P = 0.5
_DROPOUT_SEED = 1234
_GOLDEN = 0x9E3779B9


def _fused_kernel(x_ref, w_ref, b_ref, o_ref, *, n_total, threshold, seed_u,
                  scale):
    k = pl.program_id(2)
    tm, tn = o_ref.shape
    row_off = (pl.program_id(0) * tm).astype(jnp.uint32)
    col_off = (pl.program_id(1) * tn).astype(jnp.uint32)

    prod = lax.dot_general(
        x_ref[...].astype(jnp.bfloat16), w_ref[...].astype(jnp.bfloat16),
        dimension_numbers=(((1,), (1,)), ((), ())),
        preferred_element_type=jnp.float32)

    @pl.when(k == 0)
    def _():
        o_ref[...] = prod

    @pl.when(k > 0)
    def _():
        o_ref[...] += prod

    @pl.when(k == pl.num_programs(2) - 1)
    def _():
        y = jnp.maximum(o_ref[...] + b_ref[...], 0.0)
        rows = lax.broadcasted_iota(jnp.int32, (tm, tn), 0).astype(jnp.uint32) + row_off
        cols = lax.broadcasted_iota(jnp.int32, (tm, tn), 1).astype(jnp.uint32) + col_off
        idx = rows * jnp.uint32(n_total) + cols
        h = idx ^ jnp.uint32(seed_u)
        h = h ^ (h >> 16)
        h = h * jnp.uint32(0x85EBCA6B)
        h = h ^ (h >> 13)
        h = h * jnp.uint32(0xC2B2AE35)
        h = h ^ (h >> 16)
        keep = (h & jnp.uint32(0x00FFFFFF)) >= jnp.uint32(threshold)
        o_ref[...] = jnp.where(keep, y * jnp.float32(scale), 0.0)


def kernel(x, w, b):
    B, K = x.shape
    N, Kw = w.shape
    assert Kw == K

    bm = min(2048, B)
    bn = min(2048, N)
    bk = min(512, K)
    grid = (B // bm, N // bn, K // bk)

    b2 = b.reshape(1, N).astype(jnp.float32)

    seed_u = (_DROPOUT_SEED * _GOLDEN) & 0xFFFFFFFF
    threshold = int(_DROPOUT_P * (1 << 24))
    body = functools.partial(
        _fused_kernel, n_total=N, threshold=threshold, seed_u=seed_u,
        scale=1.0 / (1.0 - _DROPOUT_P))

    out = pl.pallas_call(
        body,
        grid=grid,
        in_specs=[
            pl.BlockSpec((bm, bk), lambda i, j, k: (i, k)),
            pl.BlockSpec((bn, bk), lambda i, j, k: (j, k)),
            pl.BlockSpec((1, bn), lambda i, j, k: (0, j)),
        ],
        out_specs=pl.BlockSpec((bm, bn), lambda i, j, k: (i, j)),
        out_shape=jax.ShapeDtypeStruct((B, N), jnp.float32),
        compiler_params=pltpu.CompilerParams(
            dimension_semantics=("parallel", "parallel", "arbitrary"),
            vmem_limit_bytes=64 * 1024 * 1024),
    )(x, w, b2)
    return out


# speedup vs baseline: 1.1080x; 1.1080x over previous
import functools

import jax
import jax.numpy as jnp
from jax import lax
from jax.experimental import pallas as pl
from jax.experimental.pallas import tpu as pltpu

_DROPOUT_P = 0.5
_DROPOUT_SEED = 1234
_GOLDEN = 0x9E3779B9


def _cast_kernel(x_ref, w_ref, xo_ref, wo_ref):
    xo_ref[...] = x_ref[...].astype(jnp.bfloat16)
    wo_ref[...] = w_ref[...].astype(jnp.bfloat16)


def _fused_kernel(x_ref, w_ref, b_ref, o_ref, *, n_total, seed_u, scale):
    tm, tn = o_ref.shape
    acc = lax.dot_general(
        x_ref[...], w_ref[...],
        dimension_numbers=(((1,), (1,)), ((), ())),
        preferred_element_type=jnp.float32)
    y = jnp.maximum(acc + b_ref[...], 0.0)

    base = ((pl.program_id(0) * tm) * n_total + pl.program_id(1) * tn).astype(jnp.uint32)
    lin = (lax.broadcasted_iota(jnp.int32, (tm, tn), 0) * n_total
           + lax.broadcasted_iota(jnp.int32, (tm, tn), 1)).astype(jnp.uint32)
    h = (lin + base) ^ jnp.uint32(seed_u)
    h = h ^ (h >> 16)
    h = h * jnp.uint32(0x85EBCA6B)
    h = h ^ (h >> 13)
    h = h * jnp.uint32(0xC2B2AE35)
    keep = (h & jnp.uint32(0x00800000)) != 0
    o_ref[...] = jnp.where(keep, y * jnp.float32(scale), 0.0)


def kernel(x, w, b):
    B, K = x.shape
    N, Kw = w.shape
    assert Kw == K

    bm = min(1024, B)
    bn = min(1024, N)
    grid = (B // bm, N // bn)

    g = 8
    cmx = B // g
    cmw = N // g
    xb, wb = pl.pallas_call(
        _cast_kernel,
        grid=(g,),
        in_specs=[
            pl.BlockSpec((cmx, K), lambda i: (i, 0)),
            pl.BlockSpec((cmw, K), lambda i: (i, 0)),
        ],
        out_specs=[
            pl.BlockSpec((cmx, K), lambda i: (i, 0)),
            pl.BlockSpec((cmw, K), lambda i: (i, 0)),
        ],
        out_shape=[
            jax.ShapeDtypeStruct((B, K), jnp.bfloat16),
            jax.ShapeDtypeStruct((N, K), jnp.bfloat16),
        ],
        compiler_params=pltpu.CompilerParams(
            dimension_semantics=("parallel",),
            vmem_limit_bytes=56 * 1024 * 1024),
    )(x, w)
    b2 = b.reshape(1, N).astype(jnp.float32)

    seed_u = (_DROPOUT_SEED * _GOLDEN) & 0xFFFFFFFF
    body = functools.partial(
        _fused_kernel, n_total=N, seed_u=seed_u,
        scale=1.0 / (1.0 - _DROPOUT_P))

    out = pl.pallas_call(
        body,
        grid=grid,
        in_specs=[
            pl.BlockSpec((bm, K), lambda i, j: (i, 0)),
            pl.BlockSpec((bn, K), lambda i, j: (j, 0)),
            pl.BlockSpec((1, bn), lambda i, j: (0, j)),
        ],
        out_specs=pl.BlockSpec((bm, bn), lambda i, j: (i, j)),
        out_shape=jax.ShapeDtypeStruct((B, N), jnp.float32),
        compiler_params=pltpu.CompilerParams(
            dimension_semantics=("parallel", "parallel"),
            vmem_limit_bytes=56 * 1024 * 1024),
    )(xb, wb, b2)
    return out


# speedup vs baseline: 1.2620x; 1.1390x over previous
import functools

import jax
import jax.numpy as jnp
from jax import lax
from jax.experimental import pallas as pl
from jax.experimental.pallas import tpu as pltpu

_DROPOUT_P = 0.5
_DROPOUT_SEED = 1234
_GOLDEN = 0x9E3779B9


def _fused_kernel(x_ref, w_ref, b_ref, o_ref, xb_ref, *, n_total, seed_u,
                  scale):
    j = pl.program_id(1)
    tm, tn = o_ref.shape

    @pl.when(j == 0)
    def _():
        xb_ref[...] = x_ref[...].astype(jnp.bfloat16)

    acc = lax.dot_general(
        xb_ref[...], w_ref[...].astype(jnp.bfloat16),
        dimension_numbers=(((1,), (1,)), ((), ())),
        preferred_element_type=jnp.float32)
    y = jnp.maximum(acc + b_ref[...], 0.0)

    base = ((pl.program_id(0) * tm) * n_total + j * tn).astype(jnp.uint32)
    lin = (lax.broadcasted_iota(jnp.int32, (tm, tn), 0) * n_total
           + lax.broadcasted_iota(jnp.int32, (tm, tn), 1)).astype(jnp.uint32)
    h = (lin + base) ^ jnp.uint32(seed_u)
    h = h ^ (h >> 16)
    h = h * jnp.uint32(0x85EBCA6B)
    h = h ^ (h >> 13)
    h = h * jnp.uint32(0xC2B2AE35)
    keep = (h & jnp.uint32(0x00800000)) != 0
    o_ref[...] = jnp.where(keep, y * jnp.float32(scale), 0.0)


def kernel(x, w, b):
    B, K = x.shape
    N, Kw = w.shape
    assert Kw == K

    bm = min(1024, B)
    bn = min(512, N)
    grid = (B // bm, N // bn)

    b2 = b.reshape(1, N).astype(jnp.float32)

    seed_u = (_DROPOUT_SEED * _GOLDEN) & 0xFFFFFFFF
    body = functools.partial(
        _fused_kernel, n_total=N, seed_u=seed_u,
        scale=1.0 / (1.0 - _DROPOUT_P))

    out = pl.pallas_call(
        body,
        grid=grid,
        in_specs=[
            pl.BlockSpec((bm, K), lambda i, j: (i, 0)),
            pl.BlockSpec((bn, K), lambda i, j: (j, 0)),
            pl.BlockSpec((1, bn), lambda i, j: (0, j)),
        ],
        out_specs=pl.BlockSpec((bm, bn), lambda i, j: (i, j)),
        out_shape=jax.ShapeDtypeStruct((B, N), jnp.float32),
        scratch_shapes=[pltpu.VMEM((bm, K), jnp.bfloat16)],
        compiler_params=pltpu.CompilerParams(
            dimension_semantics=("arbitrary", "arbitrary"),
            vmem_limit_bytes=64 * 1024 * 1024),
    )(x, w, b2)
    return out


# speedup vs baseline: 1.2759x; 1.0110x over previous
import functools

import jax
import jax.numpy as jnp
from jax import lax
from jax.experimental import pallas as pl
from jax.experimental.pallas import tpu as pltpu

_DROPOUT_P = 0.5
_DROPOUT_SEED = 1234
_GOLDEN = 0x9E3779B9


def _fused_kernel(x_ref, w_ref, b_ref, o_ref, xb_ref, lin_ref, *, n_total,
                  seed_u, scale):
    j = pl.program_id(1)
    tm, tn = o_ref.shape
    first = jnp.logical_and(pl.program_id(0) == 0, j == 0)

    @pl.when(first)
    def _():
        lin_ref[...] = (lax.broadcasted_iota(jnp.int32, (tm, tn), 0) * n_total
                        + lax.broadcasted_iota(jnp.int32, (tm, tn), 1)
                        ).astype(jnp.uint32)

    @pl.when(j == 0)
    def _():
        xb_ref[...] = x_ref[...].astype(jnp.bfloat16)

    base = ((pl.program_id(0) * tm) * n_total + j * tn).astype(jnp.uint32)
    xb = xb_ref[...]
    half = tn // 2
    for h0 in range(2):
        sl = pl.ds(h0 * half, half)
        acc = lax.dot_general(
            xb, w_ref[sl, :].astype(jnp.bfloat16),
            dimension_numbers=(((1,), (1,)), ((), ())),
            preferred_element_type=jnp.float32)
        y = jnp.maximum(acc + b_ref[:, sl], 0.0)
        h = (lin_ref[:, sl] + base) ^ jnp.uint32(seed_u)
        h = h ^ (h >> 16)
        h = h * jnp.uint32(0x85EBCA6B)
        h = h ^ (h >> 13)
        h = h * jnp.uint32(0xC2B2AE35)
        keep = (h & jnp.uint32(0x00800000)) != 0
        o_ref[:, sl] = jnp.where(keep, y * jnp.float32(scale), 0.0)


def kernel(x, w, b):
    B, K = x.shape
    N, Kw = w.shape
    assert Kw == K

    bm = min(1024, B)
    bn = min(512, N)
    grid = (B // bm, N // bn)

    b2 = b.reshape(1, N).astype(jnp.float32)

    seed_u = (_DROPOUT_SEED * _GOLDEN) & 0xFFFFFFFF
    body = functools.partial(
        _fused_kernel, n_total=N, seed_u=seed_u,
        scale=1.0 / (1.0 - _DROPOUT_P))

    out = pl.pallas_call(
        body,
        grid=grid,
        in_specs=[
            pl.BlockSpec((bm, K), lambda i, j: (i, 0)),
            pl.BlockSpec((bn, K), lambda i, j: (j, 0)),
            pl.BlockSpec((1, bn), lambda i, j: (0, j)),
        ],
        out_specs=pl.BlockSpec((bm, bn), lambda i, j: (i, j)),
        out_shape=jax.ShapeDtypeStruct((B, N), jnp.float32),
        scratch_shapes=[
            pltpu.VMEM((bm, K), jnp.bfloat16),
            pltpu.VMEM((bm, bn), jnp.uint32),
        ],
        compiler_params=pltpu.CompilerParams(
            dimension_semantics=("arbitrary", "arbitrary"),
            vmem_limit_bytes=64 * 1024 * 1024),
    )(x, w, b2)
    return out
